# Initial kernel scaffold; baseline (speedup 1.0000x reference)
import jax
import jax.numpy as jnp
from jax.experimental import pallas as pl


def kernel(conv1_wm, conv1_b, conv2_w, conv2_b, conv3_w, conv3_b, conv4_w, conv4_b, conv5_w, conv5_b, fc1_w, fc1_b, fc2_w, fc2_b, fc3_w, fc3_b, x_nchw):
    raise NotImplementedError("write your pallas kernel here")


# speedup vs baseline: 14.2258x; 14.2258x over previous
import functools

import jax
import jax.numpy as jnp
from jax.experimental import pallas as pl
from jax.experimental.pallas import tpu as pltpu

f32 = jnp.float32
bf16 = jnp.bfloat16


def _maxpool_val(x, Ho, Wo, ksize=3, stride=2):
    Hm = stride * (Ho - 1) + 1
    Wm = stride * (Wo - 1) + 1
    hmax = x[0:Hm]
    for k in range(1, ksize):
        hmax = jnp.maximum(hmax, x[k:k + Hm])
    wmax = hmax[:, 0:Wm]
    for k in range(1, ksize):
        wmax = jnp.maximum(wmax, hmax[:, k:k + Wm])
    rows = jnp.concatenate(
        [wmax[stride * i:stride * i + 1] for i in range(Ho)], axis=0)
    return jnp.concatenate(
        [rows[:, stride * j:stride * j + 1, :] for j in range(Wo)], axis=1)


def _im2col_mm(xflat, Wrow, M, KH, KW, wm, b):
    cols = []
    for kh in range(KH):
        for kw in range(KW):
            off = kh * Wrow + kw
            cols.append(xflat[off:off + M, :])
    patches = jnp.concatenate(cols, axis=-1)
    acc = jnp.dot(patches, wm, preferred_element_type=f32)
    return jnp.maximum(acc + b, 0.0)


def _stage1_kernel(x_ref, w_ref, b_ref, o_ref):
    out = _im2col_mm(x_ref[0], 64, 55 * 64, 3, 3, w_ref[...], b_ref[...])
    out = out.reshape(55, 64, 64)[:, :55, :]
    p = _maxpool_val(out, 27, 27)
    o_ref[...] = jnp.zeros_like(o_ref)
    o_ref[0, 2:29, 2:29, :] = p.astype(o_ref.dtype)


def _stage2_kernel(x_ref, w_ref, b_ref, o_ref):
    out = _im2col_mm(x_ref[0], 32, 27 * 32, 5, 5, w_ref[...], b_ref[...])
    out = out.reshape(27, 32, 192)[:, :27, :]
    p = _maxpool_val(out, 13, 13)
    o_ref[...] = jnp.zeros_like(o_ref)
    o_ref[0, 1:14, 1:14, :] = p.astype(o_ref.dtype)


def _stage3_kernel(x_ref, w3_ref, b3_ref, w4_ref, b4_ref, w5_ref, b5_ref,
                   o_ref, s4_ref, s5_ref):
    o3 = _im2col_mm(x_ref[0], 16, 13 * 16, 3, 3, w3_ref[...], b3_ref[...])
    o3 = o3.reshape(13, 16, 384)[:, :13, :]
    s4_ref[...] = jnp.zeros_like(s4_ref)
    s4_ref[1:14, 1:14, :] = o3.astype(bf16)

    o4 = _im2col_mm(s4_ref[...].reshape(256, 384), 16, 13 * 16, 3, 3,
                    w4_ref[...], b4_ref[...])
    o4 = o4.reshape(13, 16, 256)[:, :13, :]
    s5_ref[...] = jnp.zeros_like(s5_ref)
    s5_ref[1:14, 1:14, :] = o4.astype(bf16)

    o5 = _im2col_mm(s5_ref[...].reshape(256, 256), 16, 13 * 16, 3, 3,
                    w5_ref[...], b5_ref[...])
    o5 = o5.reshape(13, 16, 256)[:, :13, :]
    p = _maxpool_val(o5, 6, 6)
    o_ref[0] = p.reshape(36, 256).astype(o_ref.dtype)


def _conv_stage(body, x, weights, out_shape, out_dtype, scratch_shapes=()):
    N = x.shape[0]
    in_specs = [pl.BlockSpec((1,) + x.shape[1:], lambda n: (n,) + (0,) * (x.ndim - 1))]
    for wv in weights:
        in_specs.append(pl.BlockSpec(wv.shape, lambda n, _nd=wv.ndim: (0,) * _nd))
    return pl.pallas_call(
        body,
        out_shape=jax.ShapeDtypeStruct((N,) + out_shape, out_dtype),
        grid_spec=pltpu.PrefetchScalarGridSpec(
            num_scalar_prefetch=0,
            grid=(N,),
            in_specs=in_specs,
            out_specs=pl.BlockSpec(
                (1,) + out_shape, lambda n: (n,) + (0,) * len(out_shape)),
            scratch_shapes=list(scratch_shapes),
        ),
        compiler_params=pltpu.CompilerParams(
            dimension_semantics=("parallel",),
            vmem_limit_bytes=48 * 1024 * 1024,
        ),
    )(x, *weights)


def _fc_kernel(x_ref, w_ref, b_ref, o_ref, acc_ref, *, relu):
    @pl.when(pl.program_id(1) == 0)
    def _():
        acc_ref[...] = jnp.zeros_like(acc_ref)

    acc_ref[...] += jnp.dot(x_ref[...], w_ref[...],
                            preferred_element_type=f32)

    @pl.when(pl.program_id(1) == pl.num_programs(1) - 1)
    def _():
        out = acc_ref[...] + b_ref[...]
        if relu:
            out = jnp.maximum(out, 0.0)
        o_ref[...] = out.astype(o_ref.dtype)


def _fc(x, w, b, tn, tk, relu, out_dtype):
    M, K = x.shape
    _, N = w.shape
    grid = (N // tn, K // tk)
    return pl.pallas_call(
        functools.partial(_fc_kernel, relu=relu),
        out_shape=jax.ShapeDtypeStruct((M, N), out_dtype),
        grid_spec=pltpu.PrefetchScalarGridSpec(
            num_scalar_prefetch=0,
            grid=grid,
            in_specs=[
                pl.BlockSpec((M, tk), lambda j, k: (0, k)),
                pl.BlockSpec((tk, tn), lambda j, k: (k, j)),
                pl.BlockSpec((1, tn), lambda j, k: (0, j)),
            ],
            out_specs=pl.BlockSpec((M, tn), lambda j, k: (0, j)),
            scratch_shapes=[pltpu.VMEM((M, tn), f32)],
        ),
        compiler_params=pltpu.CompilerParams(
            dimension_semantics=("parallel", "arbitrary"),
            vmem_limit_bytes=48 * 1024 * 1024,
        ),
        cost_estimate=pl.CostEstimate(
            flops=2 * M * K * N, transcendentals=0,
            bytes_accessed=M * K * 2 + K * N * 2 + M * N * 4),
    )(x, w, b)


def _pack_conv1_input(x_nchw):
    N = x_nchw.shape[0]
    x = jnp.transpose(x_nchw, (0, 2, 3, 1))
    xp = jnp.pad(x, ((0, 0), (2, 2), (2, 2), (0, 0)))
    xp = xp.reshape(N, 57, 4, 57, 4, 3)
    xp = xp.transpose(0, 1, 3, 2, 4, 5).reshape(N, 57, 57, 48)
    xp = jnp.pad(xp, ((0, 0), (0, 1), (0, 7), (0, 0)))
    return xp.reshape(N, 58 * 64, 48).astype(bf16)


def _pack_conv1_weight(conv1_wm):
    w = conv1_wm[:363, :64].astype(f32).reshape(11, 11, 3, 64)
    w = jnp.pad(w, ((0, 1), (0, 1), (0, 0), (0, 0)))
    w = w.reshape(3, 4, 3, 4, 3, 64).transpose(0, 2, 1, 3, 4, 5)
    return w.reshape(9 * 48, 64).astype(bf16)


def kernel(conv1_wm, conv1_b, conv2_w, conv2_b, conv3_w, conv3_b, conv4_w,
           conv4_b, conv5_w, conv5_b, fc1_w, fc1_b, fc2_w, fc2_b, fc3_w,
           fc3_b, x_nchw):
    N = x_nchw.shape[0]

    w1m = _pack_conv1_weight(conv1_wm)
    w2m = conv2_w.reshape(5 * 5 * 64, 192)
    w3m = conv3_w.reshape(3 * 3 * 192, 384)
    w4m = conv4_w.reshape(3 * 3 * 384, 256)
    w5m = conv5_w.reshape(3 * 3 * 256, 256)
    b1 = conv1_b.astype(f32).reshape(1, 64)
    b2 = conv2_b.astype(f32).reshape(1, 192)
    b3 = conv3_b.astype(f32).reshape(1, 384)
    b4 = conv4_b.astype(f32).reshape(1, 256)
    b5 = conv5_b.astype(f32).reshape(1, 256)

    x1 = _pack_conv1_input(x_nchw)

    p1 = _conv_stage(_stage1_kernel, x1, (w1m, b1), (32, 32, 64), bf16)
    p2 = _conv_stage(_stage2_kernel, p1.reshape(N, 1024, 64), (w2m, b2),
                     (16, 16, 192), bf16)
    feats = _conv_stage(
        _stage3_kernel, p2.reshape(N, 256, 192),
        (w3m, b3, w4m, b4, w5m, b5), (36, 256), bf16,
        scratch_shapes=(pltpu.VMEM((16, 16, 384), bf16),
                        pltpu.VMEM((16, 16, 256), bf16)))

    x = feats.reshape(N, 9216)
    x = _fc(x, fc1_w, fc1_b.reshape(1, 4096), tn=512, tk=1024,
            relu=True, out_dtype=bf16)
    x = _fc(x, fc2_w, fc2_b.reshape(1, 4096), tn=512, tk=1024,
            relu=True, out_dtype=bf16)
    b3p = jnp.pad(fc3_b.astype(f32), (0, 24)).reshape(1, 1024)
    x = _fc(x, fc3_w, b3p, tn=512, tk=1024, relu=False, out_dtype=f32)
    return x[:, :1000]
